# baseline (speedup 1.0000x reference)
import functools

import jax
import jax.numpy as jnp
from jax import lax
from jax.experimental import pallas as pl
from jax.experimental.pallas import tpu as pltpu
from jax.experimental.pallas import tpu_sc as plsc

B = 1024
L = 200
D = 128
N_SEG = 3

NC, NS = 2, 16
NW = NC * NS
TOT = B * L
ROWS_W = TOT // NW
CH = 64
NCHUNK = ROWS_W // CH
NBUF = 8
LEAD = NBUF // 2


def _sc_embed(token_table, pe2d, seg_table, tok_idx, seg_idx):
    mesh = plsc.VectorSubcoreMesh(core_axis_name="c", subcore_axis_name="s")

    @functools.partial(
        pl.kernel,
        out_type=jax.ShapeDtypeStruct((TOT, D), jnp.float32),
        mesh=mesh,
        scratch_types=[
            pltpu.VMEM((L, D), jnp.float32),
            pltpu.VMEM((N_SEG, D), jnp.float32),
            pltpu.VMEM((NCHUNK, CH), jnp.int32),
            pltpu.VMEM((NCHUNK, CH), jnp.int32),
        ] + [pltpu.VMEM((CH, D), jnp.float32)] * NBUF
          + [pltpu.SemaphoreType.DMA] * (2 * NBUF),
    )
    def k(tok_hbm, pe_hbm, seg_hbm, tidx_hbm, sidx_hbm, out_hbm,
          pe_v, seg_v, tidx_v, sidx_v, *bufs_and_sems):
        bufs = bufs_and_sems[:NBUF]
        sgs = bufs_and_sems[NBUF:2 * NBUF]
        sss = bufs_and_sems[2 * NBUF:]
        wid = lax.axis_index("s") * NC + lax.axis_index("c")
        pltpu.sync_copy(pe_hbm, pe_v)
        pltpu.sync_copy(seg_hbm, seg_v)
        pltpu.sync_copy(tidx_hbm.at[wid], tidx_v)
        pltpu.sync_copy(sidx_hbm.at[wid], sidx_v)
        base = wid * ROWS_W

        def gather(c, b):
            pltpu.async_copy(tok_hbm.at[tidx_v.at[c]], bufs[b], sgs[b])

        def swait(c, b):
            pltpu.make_async_copy(
                bufs[b], out_hbm.at[pl.ds(base + c * CH, CH)], sss[b]).wait()

        def chunk(c, b, prefetch):
            bn = (b + LEAD) % NBUF
            if prefetch == "first":
                gather(c + LEAD, bn)
            elif prefetch == "steady":
                swait(c - LEAD, bn)
                gather(c + LEAD, bn)
            pltpu.make_async_copy(
                tok_hbm.at[tidx_v.at[c]], bufs[b], sgs[b]).wait()

            def row_grp(q, _):
                siv = sidx_v[c, pl.ds(16 * q, 16)]
                l0 = lax.rem(c * CH + 16 * q, L)
                sgv = [[seg_v[s, pl.ds(16 * g, 16)] for g in range(D // 16)]
                       for s in range(N_SEG)]
                for j in range(16):
                    sj = siv[j]
                    w = l0 + j
                    lj = jnp.where(w >= L, w - L, w)
                    r = 16 * q + j
                    vals = [pe_v[lj, pl.ds(16 * g, 16)]
                            + jnp.where(sj == 1, sgv[1][g],
                                        jnp.where(sj >= 2, sgv[2][g], sgv[0][g]))
                            for g in range(D // 16)]
                    for g in range(D // 16):
                        plsc.addupdate(bufs[b].at[r, pl.ds(16 * g, 16)], vals[g])
                return 0

            lax.fori_loop(0, CH // 16, row_grp, 0)
            pltpu.async_copy(
                bufs[b], out_hbm.at[pl.ds(base + c * CH, CH)], sss[b])

        for c0 in range(LEAD):
            gather(c0, c0)
        for c0 in range(LEAD):
            chunk(c0, c0, "first")
        for c0 in range(LEAD, NBUF):
            chunk(c0, c0, "steady")

        def ring_body(u, _):
            for j in range(NBUF):
                chunk(NBUF * u + NBUF + j, j, "steady")
            return 0

        n_steady = (NCHUNK - NBUF - LEAD) // NBUF
        lax.fori_loop(0, n_steady, ring_body, 0)
        for c0 in range(NCHUNK - LEAD, NCHUNK):
            chunk(c0, c0 % NBUF, "tail")
        for c0 in range(NCHUNK - NBUF, NCHUNK):
            swait(c0, c0 % NBUF)


    return k(token_table, pe2d, seg_table, tok_idx, seg_idx)


def _mask_body(seqt_ref, out_ref):
    m = (seqt_ref[...] > 0).astype(jnp.int8)
    out_ref[...] = jnp.broadcast_to(m[None, None, :, :], out_ref.shape)


def _tc_mask(seqt):
    Ib = 25
    maskt = pl.pallas_call(
        _mask_body,
        grid=(L // Ib,),
        in_specs=[pl.BlockSpec((L, B), lambda i: (0, 0))],
        out_specs=pl.BlockSpec((1, Ib, L, B), lambda i: (0, i, 0, 0)),
        out_shape=jax.ShapeDtypeStruct((1, L, L, B), jnp.int8),
    )(seqt)
    return jnp.transpose(maskt, (3, 0, 1, 2)).astype(jnp.bool_)


def kernel(input, token_table, segment_table, pe):
    seq = input[0]
    seg = input[1]
    tok_idx = seq.reshape(NW, NCHUNK, CH)
    seg_idx = seg.reshape(NW, NCHUNK, CH)
    x = _sc_embed(token_table, pe[0, :L], segment_table, tok_idx,
                  seg_idx).reshape(B, L, D)
    mask = _tc_mask(seq.T)
    return (x, mask)


# speedup vs baseline: 1.2370x; 1.2370x over previous
import functools

import jax
import jax.numpy as jnp
from jax import lax
from jax.experimental import pallas as pl
from jax.experimental.pallas import tpu as pltpu
from jax.experimental.pallas import tpu_sc as plsc

B = 1024
L = 200
D = 128
N_SEG = 3

NC, NS = 2, 16
NW = NC * NS
TOT = B * L
ROWS_W = TOT // NW
CH = 128
NCHUNK = ROWS_W // CH
NBUF = 4


def _sc_embed(token_table, pe2d, seg_table, tok_idx, seg_idx):
    mesh = plsc.VectorSubcoreMesh(core_axis_name="c", subcore_axis_name="s")

    @functools.partial(
        pl.kernel,
        out_type=jax.ShapeDtypeStruct((TOT, D), jnp.float32),
        mesh=mesh,
        scratch_types=[
            pltpu.VMEM((L, D), jnp.float32),
            pltpu.VMEM((N_SEG, D), jnp.float32),
            pltpu.VMEM((NCHUNK, CH), jnp.int32),
            pltpu.VMEM((NCHUNK, CH), jnp.int32),
        ] + [pltpu.VMEM((CH, D), jnp.float32)] * NBUF
          + [pltpu.SemaphoreType.DMA] * (2 * NBUF),
    )
    def k(tok_hbm, pe_hbm, seg_hbm, tidx_hbm, sidx_hbm, out_hbm,
          pe_v, seg_v, tidx_v, sidx_v, buf0, buf1, buf2, buf3,
          sg0, sg1, sg2, sg3, ss0, ss1, ss2, ss3):
        bufs = (buf0, buf1, buf2, buf3)
        sgs = (sg0, sg1, sg2, sg3)
        sss = (ss0, ss1, ss2, ss3)
        wid = lax.axis_index("s") * NC + lax.axis_index("c")
        pltpu.sync_copy(pe_hbm, pe_v)
        pltpu.sync_copy(seg_hbm, seg_v)
        pltpu.sync_copy(tidx_hbm.at[wid], tidx_v)
        pltpu.sync_copy(sidx_hbm.at[wid], sidx_v)
        base = wid * ROWS_W

        def gather(c, b):
            pltpu.async_copy(tok_hbm.at[tidx_v.at[c]], bufs[b], sgs[b])

        def swait(c, b):
            pltpu.make_async_copy(
                bufs[b], out_hbm.at[pl.ds(base + c * CH, CH)], sss[b]).wait()

        def chunk(c, b, prefetch):
            bn = (b + 2) % NBUF
            if prefetch == "first":
                gather(c + 2, bn)
            elif prefetch == "steady":
                swait(c - 2, bn)
                gather(c + 2, bn)
            pltpu.make_async_copy(
                tok_hbm.at[tidx_v.at[c]], bufs[b], sgs[b]).wait()

            def row_grp(q, _):
                siv = sidx_v[c, pl.ds(16 * q, 16)]
                l0 = lax.rem(c * CH + 16 * q, L)
                sgv = [[seg_v[s, pl.ds(16 * g, 16)] for g in range(D // 16)]
                       for s in range(N_SEG)]
                for j in range(16):
                    sj = siv[j]
                    w = l0 + j
                    lj = jnp.where(w >= L, w - L, w)
                    r = 16 * q + j
                    vals = [pe_v[lj, pl.ds(16 * g, 16)]
                            + jnp.where(sj == 1, sgv[1][g],
                                        jnp.where(sj >= 2, sgv[2][g], sgv[0][g]))
                            for g in range(D // 16)]
                    for g in range(D // 16):
                        plsc.addupdate(bufs[b].at[r, pl.ds(16 * g, 16)], vals[g])
                return 0

            lax.fori_loop(0, CH // 16, row_grp, 0)
            pltpu.async_copy(
                bufs[b], out_hbm.at[pl.ds(base + c * CH, CH)], sss[b])

        gather(0, 0)
        gather(1, 1)
        chunk(0, 0, "first")
        chunk(1, 1, "first")

        def quad_body(u, _):
            for j in range(NBUF):
                chunk(NBUF * u + 2 + j, (2 + j) % NBUF, "steady")
            return 0

        lax.fori_loop(0, (NCHUNK - 6) // NBUF, quad_body, 0)
        chunk(NCHUNK - 4, 2, "steady")
        chunk(NCHUNK - 3, 3, "steady")
        chunk(NCHUNK - 2, 0, "tail")
        chunk(NCHUNK - 1, 1, "tail")
        for c in range(NCHUNK - 4, NCHUNK):
            swait(c, c % NBUF)


    return k(token_table, pe2d, seg_table, tok_idx, seg_idx)


def _mask_body(seqt_ref, out_ref):
    out_ref[...] = (seqt_ref[...] > 0).astype(jnp.int8)


def _tc_mask(seqt):
    m8t = pl.pallas_call(
        _mask_body,
        in_specs=[pl.BlockSpec((L, B), lambda: (0, 0))],
        out_specs=pl.BlockSpec((L, B), lambda: (0, 0)),
        grid=(),
        out_shape=jax.ShapeDtypeStruct((L, B), jnp.int8),
    )(seqt)
    m = m8t.T.astype(jnp.bool_)
    return jnp.broadcast_to(m[:, None, None, :], (B, 1, L, L))


def kernel(input, token_table, segment_table, pe):
    seq = input[0]
    seg = input[1]
    tok_idx = seq.reshape(NW, NCHUNK, CH)
    seg_idx = seg.reshape(NW, NCHUNK, CH)
    x = _sc_embed(token_table, pe[0, :L], segment_table, tok_idx,
                  seg_idx).reshape(B, L, D)
    mask = _tc_mask(seq.T)
    return (x, mask)
